# Initial kernel scaffold; baseline (speedup 1.0000x reference)
import functools

import jax
import jax.numpy as jnp
from jax import lax
from jax.experimental import pallas as pl
from jax.experimental.pallas import tpu as pltpu
from jax.experimental.pallas import tpu_sc as plsc

N_NODES = 10000
D = 128
N_EDGES = 320000

NC = 2
NS = 16
L = 16

CHUNK = 128
NW = NC * NS
CHUNKS_PER_TILE = -(-N_EDGES // (NW * CHUNK))
E_PAD = NW * CHUNK * CHUNKS_PER_TILE
ROWS_PER_TILE = N_NODES // NS

_mesh = plsc.VectorSubcoreMesh(core_axis_name="c", subcore_axis_name="s")


@functools.partial(
    pl.kernel,
    out_type=jax.ShapeDtypeStruct((NC, N_NODES, D), jnp.float32),
    mesh=_mesh,
    scratch_types=[
        pltpu.VMEM((CHUNK,), jnp.int32),
        pltpu.VMEM((CHUNK,), jnp.int32),
        pltpu.VMEM((CHUNK,), jnp.float32),
        pltpu.VMEM((CHUNK, D), jnp.float32),
        pltpu.VMEM_SHARED((N_NODES, D), jnp.float32),
        pltpu.SemaphoreType.DMA,
    ],
)
def _layer(table_hbm, src_hbm, dst_hbm, ew_hbm, out_hbm,
           src_v, dst_v, ew_v, rows_v, acc, sem):
    c = lax.axis_index("c")
    s = lax.axis_index("s")
    wid = c * NS + s

    zeros16 = jnp.zeros((L,), jnp.float32)

    def _zrow(i, _):
        for cb in range(D // L):
            rows_v[i, pl.ds(cb * L, L)] = zeros16
        return 0
    lax.fori_loop(0, CHUNK, _zrow, 0)

    row_base = s * ROWS_PER_TILE
    for k in range(4):
        pltpu.sync_copy(rows_v, acc.at[pl.ds(row_base + k * CHUNK, CHUNK)])
    rem = ROWS_PER_TILE - 4 * CHUNK
    pltpu.sync_copy(rows_v.at[pl.ds(0, rem)],
                    acc.at[pl.ds(row_base + 4 * CHUNK, rem)])
    plsc.subcore_barrier()

    edge_base = wid * (CHUNKS_PER_TILE * CHUNK)

    def _chunk(g, _):
        off = edge_base + g * CHUNK
        pltpu.sync_copy(src_hbm.at[pl.ds(off, CHUNK)], src_v)
        pltpu.sync_copy(dst_hbm.at[pl.ds(off, CHUNK)], dst_v)
        pltpu.sync_copy(ew_hbm.at[pl.ds(off, CHUNK)], ew_v)
        pltpu.async_copy(table_hbm.at[src_v], rows_v, sem).wait()

        def _scale(r, _):
            ewb = plsc.load_gather(ew_v, [jnp.full((L,), r, jnp.int32)])
            for cb in range(D // L):
                sl = pl.ds(cb * L, L)
                rows_v[r, sl] = rows_v[r, sl] * ewb
            return 0
        lax.fori_loop(0, CHUNK, _scale, 0)

        pltpu.sync_copy(rows_v, acc.at[dst_v], add=True)
        return 0

    lax.fori_loop(0, CHUNKS_PER_TILE, _chunk, 0)
    plsc.subcore_barrier()

    pltpu.sync_copy(acc.at[pl.ds(row_base, ROWS_PER_TILE)],
                    out_hbm.at[c, pl.ds(row_base, ROWS_PER_TILE)])


def _sum2_body(p_ref, o_ref):
    o_ref[...] = p_ref[0] + p_ref[1]


def _final_body(x_ref, h1_ref, q_ref, o_ref):
    o_ref[...] = (x_ref[...] + h1_ref[...] + q_ref[0] + q_ref[1]) * (1.0 / 3.0)


_RB = 2000

_sum2 = pl.pallas_call(
    _sum2_body,
    grid=(N_NODES // _RB,),
    in_specs=[pl.BlockSpec((NC, _RB, D), lambda i: (0, i, 0))],
    out_specs=pl.BlockSpec((_RB, D), lambda i: (i, 0)),
    out_shape=jax.ShapeDtypeStruct((N_NODES, D), jnp.float32),
)

_final = pl.pallas_call(
    _final_body,
    grid=(N_NODES // _RB,),
    in_specs=[
        pl.BlockSpec((_RB, D), lambda i: (i, 0)),
        pl.BlockSpec((_RB, D), lambda i: (i, 0)),
        pl.BlockSpec((NC, _RB, D), lambda i: (0, i, 0)),
    ],
    out_specs=pl.BlockSpec((_RB, D), lambda i: (i, 0)),
    out_shape=jax.ShapeDtypeStruct((N_NODES, D), jnp.float32),
)


@jax.jit
def kernel(x, edge_index, edge_weight):
    src = edge_index[0].astype(jnp.int32)
    dst = edge_index[1].astype(jnp.int32)
    ew = edge_weight.astype(jnp.float32)

    pad = E_PAD - N_EDGES
    src = jnp.concatenate([src, jnp.zeros((pad,), jnp.int32)])
    dst = jnp.concatenate([dst, jnp.zeros((pad,), jnp.int32)])
    ew = jnp.concatenate([ew, jnp.zeros((pad,), jnp.float32)])

    p = _layer(x, src, dst, ew)
    h1 = _sum2(p)
    q = _layer(h1, src, dst, ew)
    return _final(x, h1, q)


# speedup vs baseline: 3.4291x; 3.4291x over previous
import functools

import jax
import jax.numpy as jnp
import numpy as np
from jax import lax
from jax.experimental import pallas as pl
from jax.experimental.pallas import tpu as pltpu
from jax.experimental.pallas import tpu_sc as plsc

N_NODES = 10000
D = 128
N_EDGES = 320000
N_PAD = 10240

NC = 2
NS = 16
L = 16

CHUNK = 128
NW = NC * NS
CHUNKS_PER_TILE = -(-N_EDGES // (NW * CHUNK))
E_PAD = NW * CHUNK * CHUNKS_PER_TILE
ROWS_PER_TILE = N_PAD // NS

_mesh = plsc.VectorSubcoreMesh(core_axis_name="c", subcore_axis_name="s")

_GDN = lax.GatherDimensionNumbers(
    offset_dims=(), collapsed_slice_dims=(0,), start_index_map=(0,))


def _bcast_lane(vec, j):
    idx = jnp.full((L, 1), j, jnp.int32)
    return lax.gather(vec, idx, _GDN, (1,),
                      mode=lax.GatherScatterMode.PROMISE_IN_BOUNDS)


@functools.partial(
    pl.kernel,
    out_type=jax.ShapeDtypeStruct((NC, N_PAD, D), jnp.float32),
    mesh=_mesh,
    scratch_types=[
        pltpu.VMEM((CHUNK,), jnp.int32),
        pltpu.VMEM((CHUNK,), jnp.int32),
        pltpu.VMEM((CHUNK,), jnp.float32),
        pltpu.VMEM((CHUNK, D), jnp.float32),
        pltpu.VMEM_SHARED((N_PAD, D), jnp.float32),
        pltpu.SemaphoreType.DMA,
    ],
)
def _layer(table_hbm, src_hbm, dst_hbm, ew_hbm, out_hbm,
           src_v, dst_v, ew_v, rows_v, acc, sem):
    c = lax.axis_index("c")
    s = lax.axis_index("s")
    wid = c * NS + s

    zeros16 = jnp.zeros((L,), jnp.float32)

    def _zrow(i, _):
        for cb in range(D // L):
            rows_v[i, pl.ds(cb * L, L)] = zeros16
        return 0
    lax.fori_loop(0, CHUNK, _zrow, 0)

    row_base = s * ROWS_PER_TILE
    for k in range(ROWS_PER_TILE // CHUNK):
        pltpu.sync_copy(rows_v, acc.at[pl.ds(row_base + k * CHUNK, CHUNK)])
    plsc.subcore_barrier()

    edge_base = wid * (CHUNKS_PER_TILE * CHUNK)

    def _chunk(g, _):
        off = edge_base + g * CHUNK
        pltpu.sync_copy(src_hbm.at[pl.ds(off, CHUNK)], src_v)
        pltpu.sync_copy(dst_hbm.at[pl.ds(off, CHUNK)], dst_v)
        pltpu.sync_copy(ew_hbm.at[pl.ds(off, CHUNK)], ew_v)
        pltpu.async_copy(table_hbm.at[src_v], rows_v, sem).wait()

        def _scale(gg, _):
            ewv = ew_v[pl.ds(gg * L, L)]
            for j in range(L):
                ewb = _bcast_lane(ewv, j)
                r = gg * L + j
                for cb in range(D // L):
                    sl = pl.ds(cb * L, L)
                    rows_v[r, sl] = rows_v[r, sl] * ewb
            return 0
        lax.fori_loop(0, CHUNK // L, _scale, 0)

        pltpu.sync_copy(rows_v, acc.at[dst_v], add=True)
        return 0

    lax.fori_loop(0, CHUNKS_PER_TILE, _chunk, 0)
    plsc.subcore_barrier()

    pltpu.sync_copy(acc.at[pl.ds(row_base, ROWS_PER_TILE)],
                    out_hbm.at[c, pl.ds(row_base, ROWS_PER_TILE)])


def _sum2_body(p_ref, o_ref):
    o_ref[...] = p_ref[0] + p_ref[1]


def _final_body(x_ref, h1_ref, q_ref, o_ref):
    o_ref[...] = (x_ref[...] + h1_ref[...] + q_ref[0] + q_ref[1]) * (1.0 / 3.0)


_RB = 2000
_RB2 = 1280

_sum2 = pl.pallas_call(
    _sum2_body,
    grid=(N_PAD // _RB2,),
    in_specs=[pl.BlockSpec((NC, _RB2, D), lambda i: (0, i, 0))],
    out_specs=pl.BlockSpec((_RB2, D), lambda i: (i, 0)),
    out_shape=jax.ShapeDtypeStruct((N_PAD, D), jnp.float32),
)

_final = pl.pallas_call(
    _final_body,
    grid=(N_NODES // _RB,),
    in_specs=[
        pl.BlockSpec((_RB, D), lambda i: (i, 0)),
        pl.BlockSpec((_RB, D), lambda i: (i, 0)),
        pl.BlockSpec((NC, _RB, D), lambda i: (0, i, 0)),
    ],
    out_specs=pl.BlockSpec((_RB, D), lambda i: (i, 0)),
    out_shape=jax.ShapeDtypeStruct((N_NODES, D), jnp.float32),
)


@jax.jit
def kernel(x, edge_index, edge_weight):
    src = edge_index[0].astype(jnp.int32)
    dst = edge_index[1].astype(jnp.int32)
    ew = edge_weight.astype(jnp.float32)

    pad = E_PAD - N_EDGES
    src = jnp.concatenate([src, jnp.zeros((pad,), jnp.int32)])
    dst = jnp.concatenate([dst, jnp.zeros((pad,), jnp.int32)])
    ew = jnp.concatenate([ew, jnp.zeros((pad,), jnp.float32)])

    p = _layer(x, src, dst, ew)
    h1 = _sum2(p)
    q = _layer(h1, src, dst, ew)
    return _final(x, h1, q)
